# baseline (speedup 1.0000x reference)
import functools

import jax
import jax.numpy as jnp
from jax import lax
from jax.experimental import pallas as pl
from jax.experimental.pallas import tpu as pltpu
from jax.experimental.pallas import tpu_sc as plsc

TOTAL_TOK = 16384
EMB = 128

_info = plsc.get_sparse_core_info()
_NC, _NS = _info.num_cores, _info.num_subcores
_NW = _NC * _NS
_B_PER_W = TOTAL_TOK // _NW
_CH = 128
_NCH = _B_PER_W // _CH


def _gather_body(token_hbm, table_hbm, out_hbm, idx_v, *scratch):
    rows = scratch[:_NCH]
    si = scratch[_NCH:2 * _NCH]
    so = scratch[2 * _NCH:]
    wid = lax.axis_index("s") * _NC + lax.axis_index("c")
    base = wid * _B_PER_W
    pltpu.sync_copy(token_hbm.at[pl.ds(base, _B_PER_W)], idx_v)

    gathers = [
        pltpu.async_copy(
            table_hbm.at[idx_v.at[pl.ds(k * _CH, _CH)]], rows[k], si[k])
        for k in range(_NCH)
    ]
    outs = []
    for k in range(_NCH):
        gathers[k].wait()
        outs.append(pltpu.async_copy(
            rows[k], out_hbm.at[pl.ds(base + k * _CH, _CH)], so[k]))
    for c in outs:
        c.wait()


_mesh = plsc.VectorSubcoreMesh(core_axis_name="c", subcore_axis_name="s")

_gather = functools.partial(
    pl.kernel,
    mesh=_mesh,
    out_type=jax.ShapeDtypeStruct((TOTAL_TOK, EMB), jnp.float32),
    scratch_types=(
        [pltpu.VMEM((_B_PER_W,), jnp.int32)]
        + [pltpu.VMEM((_CH, EMB), jnp.float32) for _ in range(_NCH)]
        + [pltpu.SemaphoreType.DMA for _ in range(2 * _NCH)]
    ),
)(_gather_body)


@jax.jit
def kernel(token_ids, cu_seqlens, table):
    all_embs = _gather(token_ids.astype(jnp.int32), table)
    return (all_embs, cu_seqlens)


# speedup vs baseline: 1.0121x; 1.0121x over previous
import functools

import jax
import jax.numpy as jnp
from jax import lax
from jax.experimental import pallas as pl
from jax.experimental.pallas import tpu as pltpu
from jax.experimental.pallas import tpu_sc as plsc

TOTAL_TOK = 16384
EMB = 128

_info = plsc.get_sparse_core_info()
_NC, _NS = _info.num_cores, _info.num_subcores
_NW = _NC * _NS
_B_PER_W = TOTAL_TOK // _NW
_CH = 256
_NCH = _B_PER_W // _CH


def _gather_body(token_hbm, table_hbm, out_hbm, idx_v, *scratch):
    rows = scratch[:_NCH]
    si = scratch[_NCH:2 * _NCH]
    so = scratch[2 * _NCH:]
    wid = lax.axis_index("s") * _NC + lax.axis_index("c")
    base = wid * _B_PER_W
    pltpu.sync_copy(token_hbm.at[pl.ds(base, _B_PER_W)], idx_v)

    gathers = [
        pltpu.async_copy(
            table_hbm.at[idx_v.at[pl.ds(k * _CH, _CH)]], rows[k], si[k])
        for k in range(_NCH)
    ]
    outs = []
    for k in range(_NCH):
        gathers[k].wait()
        outs.append(pltpu.async_copy(
            rows[k], out_hbm.at[pl.ds(base + k * _CH, _CH)], so[k]))
    for c in outs:
        c.wait()


_mesh = plsc.VectorSubcoreMesh(core_axis_name="c", subcore_axis_name="s")

_gather = functools.partial(
    pl.kernel,
    mesh=_mesh,
    out_type=jax.ShapeDtypeStruct((TOTAL_TOK, EMB), jnp.float32),
    scratch_types=(
        [pltpu.VMEM((_B_PER_W,), jnp.int32)]
        + [pltpu.VMEM((_CH, EMB), jnp.float32) for _ in range(_NCH)]
        + [pltpu.SemaphoreType.DMA for _ in range(2 * _NCH)]
    ),
)(_gather_body)


@jax.jit
def kernel(token_ids, cu_seqlens, table):
    all_embs = _gather(token_ids.astype(jnp.int32), table)
    return (all_embs, cu_seqlens)


# speedup vs baseline: 1.0251x; 1.0128x over previous
import functools

import jax
import jax.numpy as jnp
from jax import lax
from jax.experimental import pallas as pl
from jax.experimental.pallas import tpu as pltpu
from jax.experimental.pallas import tpu_sc as plsc

TOTAL_TOK = 16384
EMB = 128

_info = plsc.get_sparse_core_info()
_NC, _NS = _info.num_cores, _info.num_subcores
_NW = _NC * _NS
_B_PER_W = TOTAL_TOK // _NW


def _gather_body(token_hbm, table_hbm, out_hbm, idx_v, rows_v):
    wid = lax.axis_index("s") * _NC + lax.axis_index("c")
    base = wid * _B_PER_W
    pltpu.sync_copy(token_hbm.at[pl.ds(base, _B_PER_W)], idx_v)
    pltpu.sync_copy(table_hbm.at[idx_v], rows_v)
    pltpu.sync_copy(rows_v, out_hbm.at[pl.ds(base, _B_PER_W)])


_mesh = plsc.VectorSubcoreMesh(core_axis_name="c", subcore_axis_name="s")

_gather = functools.partial(
    pl.kernel,
    mesh=_mesh,
    out_type=jax.ShapeDtypeStruct((TOTAL_TOK, EMB), jnp.float32),
    scratch_types=[
        pltpu.VMEM((_B_PER_W,), jnp.int32),
        pltpu.VMEM((_B_PER_W, EMB), jnp.float32),
    ],
)(_gather_body)


@jax.jit
def kernel(token_ids, cu_seqlens, table):
    all_embs = _gather(token_ids.astype(jnp.int32), table)
    return (all_embs, cu_seqlens)
